# baseline (speedup 1.0000x reference)
import functools

import jax
import jax.numpy as jnp
import numpy as np
from jax.experimental import pallas as pl
from jax.experimental.pallas import tpu as pltpu

_B, _S, _DM = 1, 4096, 2048
_H, _HKV, _DH = 16, 2, 128
_KSEL = 1228
_SBLK = 512
_NSB = _S // _SBLK
_SQRT_DH = np.sqrt(_DH)
_NEG = -1e30

_INTERP = False


def _qproj_body(hs_ref, w_ref, b_ref, cos_ref, sin_ref, o_ref):
    q = jnp.dot(hs_ref[...], w_ref[...], preferred_element_type=jnp.float32)
    q = q + b_ref[0:1, :]
    cos = cos_ref[...]
    sin = sin_ref[...]
    for h in range(q.shape[1] // _DH):
        qh = q[:, h * _DH:(h + 1) * _DH]
        rot = jnp.concatenate([-qh[:, _DH // 2:], qh[:, :_DH // 2]], axis=1)
        o_ref[:, h * _DH:(h + 1) * _DH] = (qh * cos + rot * sin).astype(jnp.bfloat16)


def _kvproj_body(hs_ref, w_ref, b_ref, cos_ref, sin_ref, o_ref):
    kv = jnp.dot(hs_ref[...], w_ref[...], preferred_element_type=jnp.float32)
    kv = kv + b_ref[0:1, :]
    cos = cos_ref[...]
    sin = sin_ref[...]
    for h in range(_HKV):
        kh = kv[:, h * _DH:(h + 1) * _DH]
        rot = jnp.concatenate([-kh[:, _DH // 2:], kh[:, :_DH // 2]], axis=1)
        o_ref[:, h * _DH:(h + 1) * _DH] = (kh * cos + rot * sin).astype(jnp.bfloat16)
    o_ref[:, _HKV * _DH:] = kv[:, _HKV * _DH:].astype(jnp.bfloat16)


def _select_body(ql_ref, ke_ref, sel_ref, bias_ref):
    ql = ql_ref[0:1, :].astype(jnp.float32)
    acc0 = jnp.zeros((1, _DH), jnp.float32)
    acc1 = jnp.zeros((1, _DH), jnp.float32)
    for h in range(_H // 2):
        acc0 = acc0 + ql[:, h * _DH:(h + 1) * _DH]
    for h in range(_H // 2, _H):
        acc1 = acc1 + ql[:, h * _DH:(h + 1) * _DH]
    a = jnp.concatenate([acc0, acc1], axis=1) * (1.0 / _H)
    ke3 = ke_ref[...].astype(jnp.float32).reshape(32, 128, _HKV * _DH)
    imp = jnp.sum(ke3 * a.reshape(1, 1, _HKV * _DH), axis=2)

    def cnt_ge(x):
        return jnp.sum((imp >= x).astype(jnp.int32))

    lo0 = jnp.min(imp)
    hi0 = jnp.max(imp) + 1.0

    def bisect(_, c):
        lo, hi = c
        mid = 0.5 * (lo + hi)
        p = cnt_ge(mid) >= _KSEL
        return jnp.where(p, mid, lo), jnp.where(p, hi, mid)

    lo, hi = jax.lax.fori_loop(0, 64, bisect, (lo0, hi0))
    thr = lo
    count_gt = jnp.sum((imp > thr).astype(jnp.int32))
    need_eq = _KSEL - count_gt
    eq = imp == thr
    flat = (jax.lax.broadcasted_iota(jnp.int32, (32, 128), 0) * 128
            + jax.lax.broadcasted_iota(jnp.int32, (32, 128), 1))

    def bisect_idx(_, c):
        l, h = c
        mid = (l + h) // 2
        cc = jnp.sum((eq & (flat <= mid)).astype(jnp.int32))
        p = cc >= need_eq
        return jnp.where(p, l, mid + 1), jnp.where(p, mid, h)

    l2, h2 = jax.lax.fori_loop(0, 13, bisect_idx, (jnp.int32(0), jnp.int32(_S - 1)))
    sel = (imp > thr) | (eq & (flat <= h2))
    sel_ref[...] = sel.astype(jnp.int32)
    bias_ref[...] = jnp.where(sel, 0.0, _NEG).astype(jnp.float32)


def _attn_body(q_ref, k_ref, v_ref, bias_ref, o_ref):
    k = k_ref[0]
    s = jax.lax.dot_general(q_ref[...], k, (((1,), (1,)), ((), ())),
                            preferred_element_type=jnp.float32) / _SQRT_DH
    s = s + bias_ref[0:1, :]
    m = jnp.max(s, axis=1, keepdims=True)
    e = jnp.exp(s - m)
    den = jnp.sum(e, axis=1, keepdims=True)
    p = (e / den).astype(jnp.bfloat16)
    o_ref[...] = jnp.dot(p, v_ref[0],
                         preferred_element_type=jnp.float32).astype(jnp.bfloat16)


def _outproj_body(a_ref, w_ref, o_ref):
    o_ref[...] = jnp.dot(a_ref[...], w_ref[...], preferred_element_type=jnp.float32)


def kernel(hidden_states, cos, sin, Wq, bq, Wk, bk, Wv, bv, Wo):
    hs = hidden_states[0].astype(jnp.bfloat16)
    cosr = cos[:, 0]
    sinr = sin[:, 0]
    bounds = [0, 16, 40, 64, 80, 104, 128]
    cos_c = jnp.concatenate(
        [cosr[i % 3, :, bounds[i]:bounds[i + 1]] for i in range(6)], axis=-1)
    sin_c = jnp.concatenate(
        [sinr[i % 3, :, bounds[i]:bounds[i + 1]] for i in range(6)], axis=-1)

    bq_b = jnp.broadcast_to(bq.reshape(1, -1), (8, _H * _DH))
    bkv_b = jnp.broadcast_to(
        jnp.concatenate([bk, bv]).reshape(1, -1), (8, 2 * _HKV * _DH))
    wq = Wq.astype(jnp.bfloat16)
    wkv = jnp.concatenate([Wk, Wv], axis=1).astype(jnp.bfloat16)
    wo = Wo.astype(jnp.bfloat16)

    q_emb = pl.pallas_call(
        _qproj_body,
        grid=(_NSB, 4),
        in_specs=[
            pl.BlockSpec((_SBLK, _DM), lambda i, j: (i, 0)),
            pl.BlockSpec((_DM, 512), lambda i, j: (0, j)),
            pl.BlockSpec((8, 512), lambda i, j: (0, j)),
            pl.BlockSpec((_SBLK, _DH), lambda i, j: (i, 0)),
            pl.BlockSpec((_SBLK, _DH), lambda i, j: (i, 0)),
        ],
        out_specs=pl.BlockSpec((_SBLK, 512), lambda i, j: (i, j)),
        out_shape=jax.ShapeDtypeStruct((_S, _H * _DH), jnp.bfloat16),
        interpret=_INTERP,
    )(hs, wq, bq_b, cos_c, sin_c)

    kv_emb = pl.pallas_call(
        _kvproj_body,
        grid=(_NSB,),
        in_specs=[
            pl.BlockSpec((_SBLK, _DM), lambda i: (i, 0)),
            pl.BlockSpec((_DM, 512), lambda i: (0, 0)),
            pl.BlockSpec((8, 512), lambda i: (0, 0)),
            pl.BlockSpec((_SBLK, _DH), lambda i: (i, 0)),
            pl.BlockSpec((_SBLK, _DH), lambda i: (i, 0)),
        ],
        out_specs=pl.BlockSpec((_SBLK, 512), lambda i: (i, 0)),
        out_shape=jax.ShapeDtypeStruct((_S, 2 * _HKV * _DH), jnp.bfloat16),
        interpret=_INTERP,
    )(hs, wkv, bkv_b, cos_c, sin_c)

    q_last = jnp.broadcast_to(q_emb[_S - 1:_S, :], (8, _H * _DH))
    k_emb = kv_emb[:, :_HKV * _DH]

    sel32, bias32 = pl.pallas_call(
        _select_body,
        in_specs=[
            pl.BlockSpec((8, _H * _DH), lambda: (0, 0)),
            pl.BlockSpec((_S, _HKV * _DH), lambda: (0, 0)),
        ],
        out_specs=[
            pl.BlockSpec((32, 128), lambda: (0, 0)),
            pl.BlockSpec((32, 128), lambda: (0, 0)),
        ],
        out_shape=[
            jax.ShapeDtypeStruct((32, 128), jnp.int32),
            jax.ShapeDtypeStruct((32, 128), jnp.float32),
        ],
        interpret=_INTERP,
    )(q_last, k_emb)

    bias_row = jnp.broadcast_to(bias32.reshape(1, _S), (8, _S))
    k_att = jnp.stack([k_emb[:, :_DH], k_emb[:, _DH:2 * _DH]])
    v_flat = kv_emb[:, _HKV * _DH:]
    v_att = jnp.stack([v_flat[:, :_DH], v_flat[:, _DH:2 * _DH]])

    attn = pl.pallas_call(
        _attn_body,
        grid=(_H, _NSB),
        in_specs=[
            pl.BlockSpec((_SBLK, _DH), lambda h, i: (i, h)),
            pl.BlockSpec((1, _S, _DH), lambda h, i: (h // (_H // _HKV), 0, 0)),
            pl.BlockSpec((1, _S, _DH), lambda h, i: (h // (_H // _HKV), 0, 0)),
            pl.BlockSpec((8, _S), lambda h, i: (0, 0)),
        ],
        out_specs=pl.BlockSpec((_SBLK, _DH), lambda h, i: (i, h)),
        out_shape=jax.ShapeDtypeStruct((_S, _H * _DH), jnp.bfloat16),
        interpret=_INTERP,
    )(q_emb, k_att, v_att, bias_row)

    out = pl.pallas_call(
        _outproj_body,
        grid=(_NSB, 4),
        in_specs=[
            pl.BlockSpec((_SBLK, _DM), lambda i, j: (i, 0)),
            pl.BlockSpec((_DM, 512), lambda i, j: (0, j)),
        ],
        out_specs=pl.BlockSpec((_SBLK, 512), lambda i, j: (i, j)),
        out_shape=jax.ShapeDtypeStruct((_S, _DM), jnp.float32),
        interpret=_INTERP,
    )(attn, wo)

    return out.reshape(_B, _S, _DM)


# speedup vs baseline: 1.7058x; 1.7058x over previous
import functools

import jax
import jax.numpy as jnp
import numpy as np
from jax import lax
from jax.experimental import pallas as pl
from jax.experimental.pallas import tpu as pltpu
from jax.experimental.pallas import tpu_sc as plsc

_B, _S, _DM = 1, 4096, 2048
_H, _HKV, _DH = 16, 2, 128
_KSEL = 1228
_SBLK = 512
_NSB = _S // _SBLK
_SQRT_DH = np.sqrt(_DH)
_NEG = -1e30

_INTERP = False


def _qproj_body(hs_ref, w_ref, b_ref, cos_ref, sin_ref, o_ref):
    q = jnp.dot(hs_ref[...], w_ref[...], preferred_element_type=jnp.float32)
    q = q + b_ref[0:1, :]
    cos = cos_ref[...]
    sin = sin_ref[...]
    for h in range(q.shape[1] // _DH):
        qh = q[:, h * _DH:(h + 1) * _DH]
        rot = jnp.concatenate([-qh[:, _DH // 2:], qh[:, :_DH // 2]], axis=1)
        o_ref[:, h * _DH:(h + 1) * _DH] = (qh * cos + rot * sin).astype(jnp.bfloat16)


def _kvproj_body(hs_ref, w_ref, b_ref, cos_ref, sin_ref, o_ref):
    kv = jnp.dot(hs_ref[...], w_ref[...], preferred_element_type=jnp.float32)
    kv = kv + b_ref[0:1, :]
    cos = cos_ref[...]
    sin = sin_ref[...]
    for h in range(_HKV):
        kh = kv[:, h * _DH:(h + 1) * _DH]
        rot = jnp.concatenate([-kh[:, _DH // 2:], kh[:, :_DH // 2]], axis=1)
        o_ref[:, h * _DH:(h + 1) * _DH] = (kh * cos + rot * sin).astype(jnp.bfloat16)
    o_ref[:, _HKV * _DH:] = kv[:, _HKV * _DH:].astype(jnp.bfloat16)


def _select_body(ql_ref, ke_ref, dest_ref):
    ql = ql_ref[0:1, :].astype(jnp.float32)
    acc0 = jnp.zeros((1, _DH), jnp.float32)
    acc1 = jnp.zeros((1, _DH), jnp.float32)
    for h in range(_H // 2):
        acc0 = acc0 + ql[:, h * _DH:(h + 1) * _DH]
    for h in range(_H // 2, _H):
        acc1 = acc1 + ql[:, h * _DH:(h + 1) * _DH]
    a = jnp.concatenate([acc0, acc1], axis=1) * (1.0 / _H)
    ke3 = ke_ref[...].astype(jnp.float32).reshape(32, 128, _HKV * _DH)
    imp = jnp.sum(ke3 * a.reshape(1, 1, _HKV * _DH), axis=2)

    def cnt_ge(x):
        return jnp.sum((imp >= x).astype(jnp.int32))

    lo0 = jnp.min(imp)
    hi0 = jnp.max(imp) + 1.0

    def bisect(_, c):
        lo, hi = c
        mid = 0.5 * (lo + hi)
        p = cnt_ge(mid) >= _KSEL
        return jnp.where(p, mid, lo), jnp.where(p, hi, mid)

    lo, hi = jax.lax.fori_loop(0, 64, bisect, (lo0, hi0))
    thr = lo
    count_gt = jnp.sum((imp > thr).astype(jnp.int32))
    need_eq = _KSEL - count_gt
    eq = imp == thr
    flat = (jax.lax.broadcasted_iota(jnp.int32, (32, 128), 0) * 128
            + jax.lax.broadcasted_iota(jnp.int32, (32, 128), 1))

    def bisect_idx(_, c):
        l, h = c
        mid = (l + h) // 2
        cc = jnp.sum((eq & (flat <= mid)).astype(jnp.int32))
        p = cc >= need_eq
        return jnp.where(p, l, mid + 1), jnp.where(p, mid, h)

    l2, h2 = jax.lax.fori_loop(0, 13, bisect_idx, (jnp.int32(0), jnp.int32(_S - 1)))
    sel = (imp > thr) | (eq & (flat <= h2))
    sel_bf = sel.astype(jnp.bfloat16)
    iu = jax.lax.broadcasted_iota(jnp.int32, (128, 128), 0)
    ju = jax.lax.broadcasted_iota(jnp.int32, (128, 128), 1)
    upper = (iu <= ju).astype(jnp.bfloat16)
    rowcs = jnp.dot(sel_bf, upper, preferred_element_type=jnp.float32)
    il = jax.lax.broadcasted_iota(jnp.int32, (32, 32), 0)
    jl = jax.lax.broadcasted_iota(jnp.int32, (32, 32), 1)
    lower = (jl < il).astype(jnp.bfloat16)
    rowtot = rowcs[:, 127:128].astype(jnp.bfloat16)
    rowoff = jnp.dot(lower, rowtot, preferred_element_type=jnp.float32)
    pos = (rowcs + rowoff).astype(jnp.int32) - sel.astype(jnp.int32)
    dest_ref[...] = jnp.where(sel, pos, _KPAD + (flat & 15))


_KPAD = 1280
_NW = 16
_RPW = _KPAD // _NW


def _sc_gather_body(dest_hbm, kv_hbm, idx_hbm, out_hbm,
                    dest_v, idx_buf, idx_v, rows_v, sem):
    wid = lax.axis_index("s")

    @pl.when(wid == 0)
    def _compact():
        pltpu.sync_copy(dest_hbm, dest_v)
        zeros = jnp.zeros((16,), jnp.int32)
        for j in range(4):
            idx_buf[pl.ds(_KPAD - 64 + j * 16, 16)] = zeros

        def step(g, carry):
            dvec = dest_v[pl.ds(g * 16, 16)]
            vals = lax.iota(jnp.int32, 16) + g * 16
            plsc.store_scatter(idx_buf, [dvec], vals)
            return carry

        lax.fori_loop(0, _S // 16, step, jnp.int32(0))
        pltpu.sync_copy(idx_buf.at[pl.ds(0, _KPAD)], idx_hbm)

    plsc.subcore_barrier()
    pltpu.sync_copy(idx_hbm.at[pl.ds(wid * _RPW, _RPW)], idx_v)
    pltpu.async_copy(kv_hbm.at[idx_v], rows_v, sem).wait()
    pltpu.sync_copy(rows_v, out_hbm.at[pl.ds(wid * _RPW, _RPW)])


def _sc_gather(dest_flat, kv_i32):
    mesh = plsc.VectorSubcoreMesh(
        core_axis_name="c", subcore_axis_name="s", num_cores=1)
    f = functools.partial(
        pl.kernel,
        out_type=[
            jax.ShapeDtypeStruct((_KPAD,), jnp.int32),
            jax.ShapeDtypeStruct((_KPAD, _HKV * _DH), jnp.int32),
        ],
        mesh=mesh,
        scratch_types=[
            pltpu.VMEM((_S,), jnp.int32),
            pltpu.VMEM((_KPAD + 16,), jnp.int32),
            pltpu.VMEM((_RPW,), jnp.int32),
            pltpu.VMEM((_RPW, _HKV * _DH), jnp.int32),
            pltpu.SemaphoreType.DMA,
        ],
        compiler_params=pltpu.CompilerParams(needs_layout_passes=False),
    )(_sc_gather_body)
    return f(dest_flat, kv_i32)


def _attn_body(q_ref, k_ref, v_ref, bias_ref, o_ref):
    k = k_ref[0]
    s = jax.lax.dot_general(q_ref[...], k, (((1,), (1,)), ((), ())),
                            preferred_element_type=jnp.float32) / _SQRT_DH
    s = s + bias_ref[0:1, :]
    m = jnp.max(s, axis=1, keepdims=True)
    e = jnp.exp(s - m)
    den = jnp.sum(e, axis=1, keepdims=True)
    p = (e / den).astype(jnp.bfloat16)
    o_ref[...] = jnp.dot(p, v_ref[0],
                         preferred_element_type=jnp.float32).astype(jnp.bfloat16)


def _outproj_body(a_ref, w_ref, o_ref):
    o_ref[...] = jnp.dot(a_ref[...], w_ref[...], preferred_element_type=jnp.float32)


def kernel(hidden_states, cos, sin, Wq, bq, Wk, bk, Wv, bv, Wo):
    hs = hidden_states[0].astype(jnp.bfloat16)
    cosr = cos[:, 0]
    sinr = sin[:, 0]
    bounds = [0, 16, 40, 64, 80, 104, 128]
    cos_c = jnp.concatenate(
        [cosr[i % 3, :, bounds[i]:bounds[i + 1]] for i in range(6)], axis=-1)
    sin_c = jnp.concatenate(
        [sinr[i % 3, :, bounds[i]:bounds[i + 1]] for i in range(6)], axis=-1)

    bq_b = jnp.broadcast_to(bq.reshape(1, -1), (8, _H * _DH))
    bkv_b = jnp.broadcast_to(
        jnp.concatenate([bk, bv]).reshape(1, -1), (8, 2 * _HKV * _DH))
    wq = Wq.astype(jnp.bfloat16)
    wkv = jnp.concatenate([Wk, Wv], axis=1).astype(jnp.bfloat16)
    wo = Wo.astype(jnp.bfloat16)

    q_emb = pl.pallas_call(
        _qproj_body,
        grid=(_NSB, 4),
        in_specs=[
            pl.BlockSpec((_SBLK, _DM), lambda i, j: (i, 0)),
            pl.BlockSpec((_DM, 512), lambda i, j: (0, j)),
            pl.BlockSpec((8, 512), lambda i, j: (0, j)),
            pl.BlockSpec((_SBLK, _DH), lambda i, j: (i, 0)),
            pl.BlockSpec((_SBLK, _DH), lambda i, j: (i, 0)),
        ],
        out_specs=pl.BlockSpec((_SBLK, 512), lambda i, j: (i, j)),
        out_shape=jax.ShapeDtypeStruct((_S, _H * _DH), jnp.bfloat16),
        interpret=_INTERP,
    )(hs, wq, bq_b, cos_c, sin_c)

    kv_emb = pl.pallas_call(
        _kvproj_body,
        grid=(_NSB,),
        in_specs=[
            pl.BlockSpec((_SBLK, _DM), lambda i: (i, 0)),
            pl.BlockSpec((_DM, 512), lambda i: (0, 0)),
            pl.BlockSpec((8, 512), lambda i: (0, 0)),
            pl.BlockSpec((_SBLK, _DH), lambda i: (i, 0)),
            pl.BlockSpec((_SBLK, _DH), lambda i: (i, 0)),
        ],
        out_specs=pl.BlockSpec((_SBLK, 512), lambda i: (i, 0)),
        out_shape=jax.ShapeDtypeStruct((_S, 2 * _HKV * _DH), jnp.bfloat16),
        interpret=_INTERP,
    )(hs, wkv, bkv_b, cos_c, sin_c)

    q_last = jnp.broadcast_to(q_emb[_S - 1:_S, :], (8, _H * _DH))
    k_emb = kv_emb[:, :_HKV * _DH]

    dest32 = pl.pallas_call(
        _select_body,
        in_specs=[
            pl.BlockSpec((8, _H * _DH), lambda: (0, 0)),
            pl.BlockSpec((_S, _HKV * _DH), lambda: (0, 0)),
        ],
        out_specs=pl.BlockSpec((32, 128), lambda: (0, 0)),
        out_shape=jax.ShapeDtypeStruct((32, 128), jnp.int32),
        interpret=_INTERP,
    )(q_last, k_emb)

    kv_i32 = lax.bitcast_convert_type(
        kv_emb.reshape(_S, _HKV * _DH, 2), jnp.int32)
    _, kv_sp_i32 = _sc_gather(dest32.reshape(_S), kv_i32)
    kv_sp = lax.bitcast_convert_type(
        kv_sp_i32, jnp.bfloat16).reshape(_KPAD, 2 * _HKV * _DH)
    k_att = jnp.stack([kv_sp[:, :_DH], kv_sp[:, _DH:2 * _DH]])
    v_att = jnp.stack([kv_sp[:, 2 * _DH:3 * _DH], kv_sp[:, 3 * _DH:]])

    col = jnp.arange(_KPAD)
    bias_row = jnp.broadcast_to(
        jnp.where(col < _KSEL, 0.0, _NEG).astype(jnp.float32).reshape(1, _KPAD),
        (8, _KPAD))

    attn = pl.pallas_call(
        _attn_body,
        grid=(_H, _NSB),
        in_specs=[
            pl.BlockSpec((_SBLK, _DH), lambda h, i: (i, h)),
            pl.BlockSpec((1, _KPAD, _DH), lambda h, i: (h // (_H // _HKV), 0, 0)),
            pl.BlockSpec((1, _KPAD, _DH), lambda h, i: (h // (_H // _HKV), 0, 0)),
            pl.BlockSpec((8, _KPAD), lambda h, i: (0, 0)),
        ],
        out_specs=pl.BlockSpec((_SBLK, _DH), lambda h, i: (i, h)),
        out_shape=jax.ShapeDtypeStruct((_S, _H * _DH), jnp.bfloat16),
        interpret=_INTERP,
    )(q_emb, k_att, v_att, bias_row)

    out = pl.pallas_call(
        _outproj_body,
        grid=(_NSB, 4),
        in_specs=[
            pl.BlockSpec((_SBLK, _DM), lambda i, j: (i, 0)),
            pl.BlockSpec((_DM, 512), lambda i, j: (0, j)),
        ],
        out_specs=pl.BlockSpec((_SBLK, 512), lambda i, j: (i, j)),
        out_shape=jax.ShapeDtypeStruct((_S, _DM), jnp.float32),
        interpret=_INTERP,
    )(attn, wo)

    return out.reshape(_B, _S, _DM)


# speedup vs baseline: 2.6971x; 1.5811x over previous
import functools

import jax
import jax.numpy as jnp
import numpy as np
from jax import lax
from jax.experimental import pallas as pl
from jax.experimental.pallas import tpu as pltpu
from jax.experimental.pallas import tpu_sc as plsc

_B, _S, _DM = 1, 4096, 2048
_H, _HKV, _DH = 16, 2, 128
_KSEL = 1228
_SBLK = 512
_NSB = _S // _SBLK
_SQRT_DH = np.sqrt(_DH)
_NEG = -1e30

_INTERP = False


def _proj_body(hs_ref, w_ref, b_ref, cos_ref, sin_ref, oq_ref, okv_ref):
    hsb = hs_ref[...].astype(jnp.bfloat16)
    full = jnp.dot(hsb, w_ref[...], preferred_element_type=jnp.float32)
    full = full + b_ref[0:1, :]
    cos = cos_ref[...]
    sin = sin_ref[...]
    for h in range(_H):
        qh = full[:, h * _DH:(h + 1) * _DH]
        rot = jnp.concatenate([-qh[:, _DH // 2:], qh[:, :_DH // 2]], axis=1)
        oq_ref[:, h * _DH:(h + 1) * _DH] = (qh * cos + rot * sin).astype(jnp.bfloat16)
    for h in range(_HKV):
        kh = full[:, (_H + h) * _DH:(_H + h + 1) * _DH]
        rot = jnp.concatenate([-kh[:, _DH // 2:], kh[:, :_DH // 2]], axis=1)
        okv_ref[:, h * _DH:(h + 1) * _DH] = (kh * cos + rot * sin).astype(jnp.bfloat16)
    okv_ref[:, _HKV * _DH:] = full[:, (_H + _HKV) * _DH:].astype(jnp.bfloat16)


def _select_body(ql_ref, ke_ref, dest_ref):
    ql = ql_ref[0:1, :].astype(jnp.float32)
    acc0 = jnp.zeros((1, _DH), jnp.float32)
    acc1 = jnp.zeros((1, _DH), jnp.float32)
    for h in range(_H // 2):
        acc0 = acc0 + ql[:, h * _DH:(h + 1) * _DH]
    for h in range(_H // 2, _H):
        acc1 = acc1 + ql[:, h * _DH:(h + 1) * _DH]
    a = jnp.concatenate([acc0, acc1], axis=1) * (1.0 / _H)
    ke3 = ke_ref[...].astype(jnp.float32).reshape(32, 128, _HKV * _DH)
    imp = jnp.sum(ke3 * a.reshape(1, 1, _HKV * _DH), axis=2)

    def cnt_ge(x):
        return jnp.sum((imp >= x).astype(jnp.int32))

    lo0 = jnp.min(imp)
    hi0 = jnp.max(imp) + 1.0

    def bisect(_, c):
        lo, hi = c
        mid = 0.5 * (lo + hi)
        p = cnt_ge(mid) >= _KSEL
        return jnp.where(p, mid, lo), jnp.where(p, hi, mid)

    lo, hi = jax.lax.fori_loop(0, 64, bisect, (lo0, hi0))
    thr = lo
    count_gt = jnp.sum((imp > thr).astype(jnp.int32))
    need_eq = _KSEL - count_gt
    eq = imp == thr
    flat = (jax.lax.broadcasted_iota(jnp.int32, (32, 128), 0) * 128
            + jax.lax.broadcasted_iota(jnp.int32, (32, 128), 1))

    def bisect_idx(_, c):
        l, h = c
        mid = (l + h) // 2
        cc = jnp.sum((eq & (flat <= mid)).astype(jnp.int32))
        p = cc >= need_eq
        return jnp.where(p, l, mid + 1), jnp.where(p, mid, h)

    l2, h2 = jax.lax.fori_loop(0, 13, bisect_idx, (jnp.int32(0), jnp.int32(_S - 1)))
    sel = (imp > thr) | (eq & (flat <= h2))
    sel_bf = sel.astype(jnp.bfloat16)
    iu = jax.lax.broadcasted_iota(jnp.int32, (128, 128), 0)
    ju = jax.lax.broadcasted_iota(jnp.int32, (128, 128), 1)
    upper = (iu <= ju).astype(jnp.bfloat16)
    rowcs = jnp.dot(sel_bf, upper, preferred_element_type=jnp.float32)
    il = jax.lax.broadcasted_iota(jnp.int32, (32, 32), 0)
    jl = jax.lax.broadcasted_iota(jnp.int32, (32, 32), 1)
    lower = (jl < il).astype(jnp.bfloat16)
    rowtot = rowcs[:, 127:128].astype(jnp.bfloat16)
    rowoff = jnp.dot(lower, rowtot, preferred_element_type=jnp.float32)
    pos = (rowcs + rowoff).astype(jnp.int32) - sel.astype(jnp.int32)
    dest_ref[...] = jnp.where(sel, pos, _KSEL + flat - pos)


_KPAD = 1280
_NW = 16
_TPW = _S // _NW
_OPAD = _S


def _sc_scatter_body(dest_hbm, kv_hbm, out_hbm, idx_v, rows_v, sem):
    wid = lax.axis_index("s")
    nch = _TPW // 128
    pltpu.sync_copy(dest_hbm.at[pl.ds(wid * nch, nch)], idx_v)
    pltpu.sync_copy(kv_hbm.at[pl.ds(wid * _TPW, _TPW)], rows_v)
    for j in range(nch):
        pltpu.async_copy(rows_v.at[pl.ds(j * 128, 128)],
                         out_hbm.at[idx_v.at[j]], sem).wait()


def _sc_gather(dest_flat, kv_i32):
    mesh = plsc.VectorSubcoreMesh(
        core_axis_name="c", subcore_axis_name="s", num_cores=1)
    f = functools.partial(
        pl.kernel,
        out_type=jax.ShapeDtypeStruct((_OPAD, _HKV * _DH), jnp.int32),
        mesh=mesh,
        scratch_types=[
            pltpu.VMEM((_TPW // 128, 128), jnp.int32),
            pltpu.VMEM((_TPW, _HKV * _DH), jnp.int32),
            pltpu.SemaphoreType.DMA,
        ],
    )(_sc_scatter_body)
    return f(dest_flat.reshape(_NW * (_TPW // 128), 128), kv_i32)


def _attn_out_body(q_ref, k_ref, v_ref, bias_ref, wo_ref, o_ref):
    bias = bias_ref[0:1, :]
    cols = []
    for g in range(_HKV):
        k = k_ref[g]
        v = v_ref[g]
        for hh in range(_H // _HKV):
            h = g * (_H // _HKV) + hh
            q = q_ref[:, h * _DH:(h + 1) * _DH]
            s = jax.lax.dot_general(q, k, (((1,), (1,)), ((), ())),
                                    preferred_element_type=jnp.float32)
            e = jnp.exp(s / _SQRT_DH + bias)
            den = jnp.sum(e, axis=1, keepdims=True)
            p = (e / den).astype(jnp.bfloat16)
            cols.append(jnp.dot(p, v, preferred_element_type=jnp.float32)
                        .astype(jnp.bfloat16))
    attn = jnp.concatenate(cols, axis=1)
    o_ref[...] = jnp.dot(attn, wo_ref[...], preferred_element_type=jnp.float32)


def kernel(hidden_states, cos, sin, Wq, bq, Wk, bk, Wv, bv, Wo):
    cosr = cos[:, 0]
    sinr = sin[:, 0]
    bounds = [0, 16, 40, 64, 80, 104, 128]
    cos_c = jnp.concatenate(
        [cosr[i % 3, :, bounds[i]:bounds[i + 1]] for i in range(6)], axis=-1)
    sin_c = jnp.concatenate(
        [sinr[i % 3, :, bounds[i]:bounds[i + 1]] for i in range(6)], axis=-1)

    ball = jnp.broadcast_to(
        jnp.concatenate([bq, bk, bv]).reshape(1, -1), (8, (_H + 2 * _HKV) * _DH))
    wall = jnp.concatenate([Wq, Wk, Wv], axis=1).astype(jnp.bfloat16)
    wo = Wo.astype(jnp.bfloat16)

    q_emb, kv_emb = pl.pallas_call(
        _proj_body,
        grid=(_NSB,),
        in_specs=[
            pl.BlockSpec((_SBLK, _DM), lambda i: (i, 0)),
            pl.BlockSpec((_DM, (_H + 2 * _HKV) * _DH), lambda i: (0, 0)),
            pl.BlockSpec((8, (_H + 2 * _HKV) * _DH), lambda i: (0, 0)),
            pl.BlockSpec((_SBLK, _DH), lambda i: (i, 0)),
            pl.BlockSpec((_SBLK, _DH), lambda i: (i, 0)),
        ],
        out_specs=[
            pl.BlockSpec((_SBLK, _H * _DH), lambda i: (i, 0)),
            pl.BlockSpec((_SBLK, 2 * _HKV * _DH), lambda i: (i, 0)),
        ],
        out_shape=[
            jax.ShapeDtypeStruct((_S, _H * _DH), jnp.bfloat16),
            jax.ShapeDtypeStruct((_S, 2 * _HKV * _DH), jnp.bfloat16),
        ],
        interpret=_INTERP,
    )(hidden_states[0], wall, ball, cos_c, sin_c)

    q_last = jnp.broadcast_to(q_emb[_S - 1:_S, :], (8, _H * _DH))
    k_emb = kv_emb[:, :_HKV * _DH]

    dest32 = pl.pallas_call(
        _select_body,
        in_specs=[
            pl.BlockSpec((8, _H * _DH), lambda: (0, 0)),
            pl.BlockSpec((_S, _HKV * _DH), lambda: (0, 0)),
        ],
        out_specs=pl.BlockSpec((32, 128), lambda: (0, 0)),
        out_shape=jax.ShapeDtypeStruct((32, 128), jnp.int32),
        interpret=_INTERP,
    )(q_last, k_emb)

    kv_i32 = lax.bitcast_convert_type(
        kv_emb.reshape(_S, _HKV * _DH, 2), jnp.int32)
    kv_sp_i32 = _sc_gather(dest32.reshape(_S), kv_i32)
    kv_sp = lax.bitcast_convert_type(
        kv_sp_i32[:_KPAD], jnp.bfloat16).reshape(_KPAD, 2 * _HKV * _DH)
    k_att = jnp.stack([kv_sp[:, :_DH], kv_sp[:, _DH:2 * _DH]])
    v_att = jnp.stack([kv_sp[:, 2 * _DH:3 * _DH], kv_sp[:, 3 * _DH:]])

    col = jnp.arange(_KPAD)
    bias_row = jnp.broadcast_to(
        jnp.where(col < _KSEL, 0.0, _NEG).astype(jnp.float32).reshape(1, _KPAD),
        (8, _KPAD))

    out = pl.pallas_call(
        _attn_out_body,
        grid=(_NSB,),
        in_specs=[
            pl.BlockSpec((_SBLK, _H * _DH), lambda i: (i, 0)),
            pl.BlockSpec((_HKV, _KPAD, _DH), lambda i: (0, 0, 0)),
            pl.BlockSpec((_HKV, _KPAD, _DH), lambda i: (0, 0, 0)),
            pl.BlockSpec((8, _KPAD), lambda i: (0, 0)),
            pl.BlockSpec((_DM, _DM), lambda i: (0, 0)),
        ],
        out_specs=pl.BlockSpec((_SBLK, _DM), lambda i: (i, 0)),
        out_shape=jax.ShapeDtypeStruct((_S, _DM), jnp.float32),
        interpret=_INTERP,
    )(q_emb, k_att, v_att, bias_row, wo)

    return out.reshape(_B, _S, _DM)


# speedup vs baseline: 2.7135x; 1.0061x over previous
import functools

import jax
import jax.numpy as jnp
import numpy as np
from jax import lax
from jax.experimental import pallas as pl
from jax.experimental.pallas import tpu as pltpu
from jax.experimental.pallas import tpu_sc as plsc

_B, _S, _DM = 1, 4096, 2048
_H, _HKV, _DH = 16, 2, 128
_KSEL = 1228
_SBLK = 512
_NSB = _S // _SBLK
_SQRT_DH = np.sqrt(_DH)
_NEG = -1e30

_INTERP = False


def _proj_body(hs_ref, wq_ref, wkv_ref, b_ref, cos_ref, sin_ref, oq_ref, okv_ref):
    hsb = hs_ref[...].astype(jnp.bfloat16)
    q = jnp.dot(hsb, wq_ref[...], preferred_element_type=jnp.float32)
    q = q + b_ref[0:1, :_H * _DH]
    kv = jnp.dot(hsb, wkv_ref[...], preferred_element_type=jnp.float32)
    kv = kv + b_ref[0:1, _H * _DH:]
    cos = cos_ref[...]
    sin = sin_ref[...]
    for h in range(_H):
        qh = q[:, h * _DH:(h + 1) * _DH]
        rot = jnp.concatenate([-qh[:, _DH // 2:], qh[:, :_DH // 2]], axis=1)
        oq_ref[:, h * _DH:(h + 1) * _DH] = (qh * cos + rot * sin).astype(jnp.bfloat16)
    for h in range(_HKV):
        kh = kv[:, h * _DH:(h + 1) * _DH]
        rot = jnp.concatenate([-kh[:, _DH // 2:], kh[:, :_DH // 2]], axis=1)
        okv_ref[:, h * _DH:(h + 1) * _DH] = (kh * cos + rot * sin).astype(jnp.bfloat16)
    okv_ref[:, _HKV * _DH:] = kv[:, _HKV * _DH:].astype(jnp.bfloat16)


def _select_body(ql_ref, ke_ref, dest_ref):
    ql = ql_ref[0:1, :].astype(jnp.float32)
    acc0 = jnp.zeros((1, _DH), jnp.float32)
    acc1 = jnp.zeros((1, _DH), jnp.float32)
    for h in range(_H // 2):
        acc0 = acc0 + ql[:, h * _DH:(h + 1) * _DH]
    for h in range(_H // 2, _H):
        acc1 = acc1 + ql[:, h * _DH:(h + 1) * _DH]
    a = jnp.concatenate([acc0, acc1], axis=1) * (1.0 / _H)
    ke3 = ke_ref[...].astype(jnp.float32).reshape(32, 128, _HKV * _DH)
    imp = jnp.sum(ke3 * a.reshape(1, 1, _HKV * _DH), axis=2)

    def cnt_ge(x):
        return jnp.sum((imp >= x).astype(jnp.int32))

    lo0 = jnp.min(imp)
    hi0 = jnp.max(imp) + 1.0

    def bisect(_, c):
        lo, hi = c
        mid = 0.5 * (lo + hi)
        p = cnt_ge(mid) >= _KSEL
        return jnp.where(p, mid, lo), jnp.where(p, hi, mid)

    lo, hi = jax.lax.fori_loop(0, 64, bisect, (lo0, hi0))
    thr = lo
    count_gt = jnp.sum((imp > thr).astype(jnp.int32))
    need_eq = _KSEL - count_gt
    eq = imp == thr
    flat = (jax.lax.broadcasted_iota(jnp.int32, (32, 128), 0) * 128
            + jax.lax.broadcasted_iota(jnp.int32, (32, 128), 1))

    def bisect_idx(_, c):
        l, h = c
        mid = (l + h) // 2
        cc = jnp.sum((eq & (flat <= mid)).astype(jnp.int32))
        p = cc >= need_eq
        return jnp.where(p, l, mid + 1), jnp.where(p, mid, h)

    l2, h2 = jax.lax.fori_loop(0, 13, bisect_idx, (jnp.int32(0), jnp.int32(_S - 1)))
    sel = (imp > thr) | (eq & (flat <= h2))
    sel_bf = sel.astype(jnp.bfloat16)
    iu = jax.lax.broadcasted_iota(jnp.int32, (128, 128), 0)
    ju = jax.lax.broadcasted_iota(jnp.int32, (128, 128), 1)
    upper = (iu <= ju).astype(jnp.bfloat16)
    rowcs = jnp.dot(sel_bf, upper, preferred_element_type=jnp.float32)
    il = jax.lax.broadcasted_iota(jnp.int32, (32, 32), 0)
    jl = jax.lax.broadcasted_iota(jnp.int32, (32, 32), 1)
    lower = (jl < il).astype(jnp.bfloat16)
    rowtot = rowcs[:, 127:128].astype(jnp.bfloat16)
    rowoff = jnp.dot(lower, rowtot, preferred_element_type=jnp.float32)
    pos = (rowcs + rowoff).astype(jnp.int32) - sel.astype(jnp.int32)
    dest_ref[...] = jnp.where(sel, pos, _KSEL + flat - pos)


_KPAD = 1280
_NW = 16
_TPW = _S // _NW
_OPAD = _S


def _sc_scatter_body(dest_hbm, kv_hbm, out_hbm, idx_v, rows_v, sem):
    wid = lax.axis_index("s")
    nch = _TPW // 128
    pltpu.sync_copy(dest_hbm.at[pl.ds(wid * nch, nch)], idx_v)
    pltpu.sync_copy(kv_hbm.at[pl.ds(wid * _TPW, _TPW)], rows_v)
    for j in range(nch):
        pltpu.async_copy(rows_v.at[pl.ds(j * 128, 128)],
                         out_hbm.at[idx_v.at[j]], sem).wait()


def _sc_gather(dest_flat, kv_i32):
    mesh = plsc.VectorSubcoreMesh(
        core_axis_name="c", subcore_axis_name="s", num_cores=1)
    f = functools.partial(
        pl.kernel,
        out_type=jax.ShapeDtypeStruct((_OPAD, _HKV * _DH), jnp.int32),
        mesh=mesh,
        scratch_types=[
            pltpu.VMEM((_TPW // 128, 128), jnp.int32),
            pltpu.VMEM((_TPW, _HKV * _DH), jnp.int32),
            pltpu.SemaphoreType.DMA,
        ],
    )(_sc_scatter_body)
    return f(dest_flat.reshape(_NW * (_TPW // 128), 128), kv_i32)


_LOG2E = float(np.log2(np.e))
_SC_EXP2 = _LOG2E / _SQRT_DH


def _attn_out_body(q_ref, kv_ref, bias_ref, wo_ref, o_ref):
    bias2 = bias_ref[0:1, :]
    cols = []
    for g in range(_HKV):
        k = kv_ref[:, g * _DH:(g + 1) * _DH]
        v = kv_ref[:, (_HKV + g) * _DH:(_HKV + g + 1) * _DH]
        for hh in range(_H // _HKV):
            h = g * (_H // _HKV) + hh
            q = q_ref[:, h * _DH:(h + 1) * _DH]
            s = jax.lax.dot_general(q, k, (((1,), (1,)), ((), ())),
                                    preferred_element_type=jnp.float32)
            e = jnp.exp2(s * _SC_EXP2 + bias2)
            den = jnp.sum(e, axis=1, keepdims=True)
            pv = jnp.dot(e.astype(jnp.bfloat16), v,
                         preferred_element_type=jnp.float32)
            cols.append((pv / den).astype(jnp.bfloat16))
    attn = jnp.concatenate(cols, axis=1)
    o_ref[...] = jnp.dot(attn, wo_ref[...], preferred_element_type=jnp.float32)


def kernel(hidden_states, cos, sin, Wq, bq, Wk, bk, Wv, bv, Wo):
    cosr = cos[:, 0]
    sinr = sin[:, 0]
    bounds = [0, 16, 40, 64, 80, 104, 128]
    cos_c = jnp.concatenate(
        [cosr[i % 3, :, bounds[i]:bounds[i + 1]] for i in range(6)], axis=-1)
    sin_c = jnp.concatenate(
        [sinr[i % 3, :, bounds[i]:bounds[i + 1]] for i in range(6)], axis=-1)

    ball = jnp.broadcast_to(
        jnp.concatenate([bq, bk, bv]).reshape(1, -1), (8, (_H + 2 * _HKV) * _DH))
    wq = Wq.astype(jnp.bfloat16)
    wkv = jnp.concatenate([Wk, Wv], axis=1).astype(jnp.bfloat16)
    wo = Wo.astype(jnp.bfloat16)

    q_emb, kv_emb = pl.pallas_call(
        _proj_body,
        grid=(_NSB,),
        in_specs=[
            pl.BlockSpec((_SBLK, _DM), lambda i: (i, 0)),
            pl.BlockSpec((_DM, _H * _DH), lambda i: (0, 0)),
            pl.BlockSpec((_DM, 2 * _HKV * _DH), lambda i: (0, 0)),
            pl.BlockSpec((8, (_H + 2 * _HKV) * _DH), lambda i: (0, 0)),
            pl.BlockSpec((_SBLK, _DH), lambda i: (i, 0)),
            pl.BlockSpec((_SBLK, _DH), lambda i: (i, 0)),
        ],
        out_specs=[
            pl.BlockSpec((_SBLK, _H * _DH), lambda i: (i, 0)),
            pl.BlockSpec((_SBLK, 2 * _HKV * _DH), lambda i: (i, 0)),
        ],
        out_shape=[
            jax.ShapeDtypeStruct((_S, _H * _DH), jnp.bfloat16),
            jax.ShapeDtypeStruct((_S, 2 * _HKV * _DH), jnp.bfloat16),
        ],
        interpret=_INTERP,
    )(hidden_states[0], wq, wkv, ball, cos_c, sin_c)

    q_last = jnp.broadcast_to(q_emb[_S - 1:_S, :], (8, _H * _DH))
    k_emb = kv_emb[:, :_HKV * _DH]

    dest32 = pl.pallas_call(
        _select_body,
        in_specs=[
            pl.BlockSpec((8, _H * _DH), lambda: (0, 0)),
            pl.BlockSpec((_S, _HKV * _DH), lambda: (0, 0)),
        ],
        out_specs=pl.BlockSpec((32, 128), lambda: (0, 0)),
        out_shape=jax.ShapeDtypeStruct((32, 128), jnp.int32),
        interpret=_INTERP,
    )(q_last, k_emb)

    kv_i32 = lax.bitcast_convert_type(
        kv_emb.reshape(_S, _HKV * _DH, 2), jnp.int32)
    kv_sp_i32 = _sc_gather(dest32.reshape(_S), kv_i32)
    kv_sp = lax.bitcast_convert_type(
        kv_sp_i32[:_KPAD], jnp.bfloat16).reshape(_KPAD, 2 * _HKV * _DH)

    col = jnp.arange(_KPAD)
    bias_row = jnp.broadcast_to(
        jnp.where(col < _KSEL, 0.0, _NEG * _LOG2E).astype(jnp.float32)
        .reshape(1, _KPAD), (8, _KPAD))

    qblk = 1024
    out = pl.pallas_call(
        _attn_out_body,
        grid=(_S // qblk,),
        in_specs=[
            pl.BlockSpec((qblk, _H * _DH), lambda i: (i, 0)),
            pl.BlockSpec((_KPAD, 2 * _HKV * _DH), lambda i: (0, 0)),
            pl.BlockSpec((8, _KPAD), lambda i: (0, 0)),
            pl.BlockSpec((_DM, _DM), lambda i: (0, 0)),
        ],
        out_specs=pl.BlockSpec((qblk, _DM), lambda i: (i, 0)),
        out_shape=jax.ShapeDtypeStruct((_S, _DM), jnp.float32),
        interpret=_INTERP,
    )(q_emb, kv_sp, bias_row, wo)

    return out.reshape(_B, _S, _DM)


# speedup vs baseline: 2.7177x; 1.0015x over previous
import functools

import jax
import jax.numpy as jnp
import numpy as np
from jax import lax
from jax.experimental import pallas as pl
from jax.experimental.pallas import tpu as pltpu
from jax.experimental.pallas import tpu_sc as plsc

_B, _S, _DM = 1, 4096, 2048
_H, _HKV, _DH = 16, 2, 128
_KSEL = 1228
_SBLK = 512
_NSB = _S // _SBLK
_SQRT_DH = np.sqrt(_DH)
_NEG = -1e30


def _proj_body(hs_ref, wq_ref, wkv_ref, b_ref, cos_ref, sin_ref, oq_ref, okv_ref):
    hsb = hs_ref[...].astype(jnp.bfloat16)
    q = jnp.dot(hsb, wq_ref[...], preferred_element_type=jnp.float32)
    q = q + b_ref[0:1, :_H * _DH]
    kv = jnp.dot(hsb, wkv_ref[...], preferred_element_type=jnp.float32)
    kv = kv + b_ref[0:1, _H * _DH:]
    cos = cos_ref[...]
    sin = sin_ref[...]
    for h in range(_H):
        qh = q[:, h * _DH:(h + 1) * _DH]
        rot = jnp.concatenate([-qh[:, _DH // 2:], qh[:, :_DH // 2]], axis=1)
        oq_ref[:, h * _DH:(h + 1) * _DH] = (qh * cos + rot * sin).astype(jnp.bfloat16)
    for h in range(_HKV):
        kh = kv[:, h * _DH:(h + 1) * _DH]
        rot = jnp.concatenate([-kh[:, _DH // 2:], kh[:, :_DH // 2]], axis=1)
        okv_ref[:, h * _DH:(h + 1) * _DH] = (kh * cos + rot * sin).astype(jnp.bfloat16)
    okv_ref[:, _HKV * _DH:] = kv[:, _HKV * _DH:].astype(jnp.bfloat16)


def _select_body(ql_ref, ke_ref, dest_ref):
    ql = ql_ref[0:1, :].astype(jnp.float32)
    acc0 = jnp.zeros((1, _DH), jnp.float32)
    acc1 = jnp.zeros((1, _DH), jnp.float32)
    for h in range(_H // 2):
        acc0 = acc0 + ql[:, h * _DH:(h + 1) * _DH]
    for h in range(_H // 2, _H):
        acc1 = acc1 + ql[:, h * _DH:(h + 1) * _DH]
    a = jnp.concatenate([acc0, acc1], axis=1) * (1.0 / _H)
    ke3 = ke_ref[...].astype(jnp.float32).reshape(32, 128, _HKV * _DH)
    imp = jnp.sum(ke3 * a.reshape(1, 1, _HKV * _DH), axis=2)

    def cnt_ge(x):
        return jnp.sum((imp >= x).astype(jnp.int32))

    lo0 = jnp.min(imp)
    hi0 = jnp.max(imp) + 1.0

    def bisect(_, c):
        lo, hi = c
        mid = 0.5 * (lo + hi)
        p = cnt_ge(mid) >= _KSEL
        return jnp.where(p, mid, lo), jnp.where(p, hi, mid)

    lo, hi = jax.lax.fori_loop(0, 64, bisect, (lo0, hi0))
    thr = lo
    count_gt = jnp.sum((imp > thr).astype(jnp.int32))
    need_eq = _KSEL - count_gt
    eq = imp == thr
    flat = (jax.lax.broadcasted_iota(jnp.int32, (32, 128), 0) * 128
            + jax.lax.broadcasted_iota(jnp.int32, (32, 128), 1))

    def bisect_idx(_, c):
        l, h = c
        mid = (l + h) // 2
        cc = jnp.sum((eq & (flat <= mid)).astype(jnp.int32))
        p = cc >= need_eq
        return jnp.where(p, l, mid + 1), jnp.where(p, mid, h)

    l2, h2 = jax.lax.fori_loop(0, 13, bisect_idx, (jnp.int32(0), jnp.int32(_S - 1)))
    sel = (imp > thr) | (eq & (flat <= h2))
    sel_bf = sel.astype(jnp.bfloat16)
    iu = jax.lax.broadcasted_iota(jnp.int32, (128, 128), 0)
    ju = jax.lax.broadcasted_iota(jnp.int32, (128, 128), 1)
    upper = (iu <= ju).astype(jnp.bfloat16)
    rowcs = jnp.dot(sel_bf, upper, preferred_element_type=jnp.float32)
    il = jax.lax.broadcasted_iota(jnp.int32, (32, 32), 0)
    jl = jax.lax.broadcasted_iota(jnp.int32, (32, 32), 1)
    lower = (jl < il).astype(jnp.bfloat16)
    rowtot = rowcs[:, 127:128].astype(jnp.bfloat16)
    rowoff = jnp.dot(lower, rowtot, preferred_element_type=jnp.float32)
    pos = (rowcs + rowoff).astype(jnp.int32) - sel.astype(jnp.int32)
    dest_ref[...] = jnp.where(sel, pos, _KSEL + flat - pos)


_KPAD = 1280
_NW = 16
_TPW = _S // _NW
_OPAD = _S


def _sc_scatter_body(dest_hbm, kv_hbm, out_hbm, idx_v, rows_v, sem):
    wid = lax.axis_index("s")
    nch = _TPW // 128
    pltpu.sync_copy(dest_hbm.at[pl.ds(wid * nch, nch)], idx_v)
    pltpu.sync_copy(kv_hbm.at[pl.ds(wid * _TPW, _TPW)], rows_v)
    for j in range(nch):
        pltpu.async_copy(rows_v.at[pl.ds(j * 128, 128)],
                         out_hbm.at[idx_v.at[j]], sem).wait()


def _sc_gather(dest_flat, kv_i32):
    mesh = plsc.VectorSubcoreMesh(
        core_axis_name="c", subcore_axis_name="s", num_cores=1)
    f = functools.partial(
        pl.kernel,
        out_type=jax.ShapeDtypeStruct((_OPAD, _HKV * _DH), jnp.int32),
        mesh=mesh,
        scratch_types=[
            pltpu.VMEM((_TPW // 128, 128), jnp.int32),
            pltpu.VMEM((_TPW, _HKV * _DH), jnp.int32),
            pltpu.SemaphoreType.DMA,
        ],
    )(_sc_scatter_body)
    return f(dest_flat.reshape(_NW * (_TPW // 128), 128), kv_i32)


_LOG2E = float(np.log2(np.e))
_SC_EXP2 = _LOG2E / _SQRT_DH


def _attn_out_body(q_ref, kv_ref, bias_ref, wo_ref, o_ref):
    bias2 = bias_ref[0:1, :]
    cols = []
    for g in range(_HKV):
        k = kv_ref[:, g * _DH:(g + 1) * _DH]
        v = kv_ref[:, (_HKV + g) * _DH:(_HKV + g + 1) * _DH]
        for hh in range(_H // _HKV):
            h = g * (_H // _HKV) + hh
            q = q_ref[:, h * _DH:(h + 1) * _DH]
            s = jax.lax.dot_general(q, k, (((1,), (1,)), ((), ())),
                                    preferred_element_type=jnp.float32)
            e = jnp.exp2(s * _SC_EXP2 + bias2)
            den = jnp.sum(e, axis=1, keepdims=True)
            pv = jnp.dot(e.astype(jnp.bfloat16), v,
                         preferred_element_type=jnp.float32)
            cols.append((pv / den).astype(jnp.bfloat16))
    attn = jnp.concatenate(cols, axis=1)
    o_ref[...] = jnp.dot(attn, wo_ref[...], preferred_element_type=jnp.float32)


def kernel(hidden_states, cos, sin, Wq, bq, Wk, bk, Wv, bv, Wo):
    cosr = cos[:, 0]
    sinr = sin[:, 0]
    bounds = [0, 16, 40, 64, 80, 104, 128]
    cos_c = jnp.concatenate(
        [cosr[i % 3, :, bounds[i]:bounds[i + 1]] for i in range(6)], axis=-1)
    sin_c = jnp.concatenate(
        [sinr[i % 3, :, bounds[i]:bounds[i + 1]] for i in range(6)], axis=-1)

    ball = jnp.broadcast_to(
        jnp.concatenate([bq, bk, bv]).reshape(1, -1), (8, (_H + 2 * _HKV) * _DH))
    wq = Wq.astype(jnp.bfloat16)
    wkv = jnp.concatenate([Wk, Wv], axis=1).astype(jnp.bfloat16)
    wo = Wo.astype(jnp.bfloat16)

    q_emb, kv_emb = pl.pallas_call(
        _proj_body,
        grid=(_NSB,),
        in_specs=[
            pl.BlockSpec((_SBLK, _DM), lambda i: (i, 0)),
            pl.BlockSpec((_DM, _H * _DH), lambda i: (0, 0)),
            pl.BlockSpec((_DM, 2 * _HKV * _DH), lambda i: (0, 0)),
            pl.BlockSpec((8, (_H + 2 * _HKV) * _DH), lambda i: (0, 0)),
            pl.BlockSpec((_SBLK, _DH), lambda i: (i, 0)),
            pl.BlockSpec((_SBLK, _DH), lambda i: (i, 0)),
        ],
        out_specs=[
            pl.BlockSpec((_SBLK, _H * _DH), lambda i: (i, 0)),
            pl.BlockSpec((_SBLK, 2 * _HKV * _DH), lambda i: (i, 0)),
        ],
        out_shape=[
            jax.ShapeDtypeStruct((_S, _H * _DH), jnp.bfloat16),
            jax.ShapeDtypeStruct((_S, 2 * _HKV * _DH), jnp.bfloat16),
        ],
    )(hidden_states[0], wq, wkv, ball, cos_c, sin_c)

    q_last = jnp.broadcast_to(q_emb[_S - 1:_S, :], (8, _H * _DH))
    k_emb = kv_emb[:, :_HKV * _DH]

    dest32 = pl.pallas_call(
        _select_body,
        in_specs=[
            pl.BlockSpec((8, _H * _DH), lambda: (0, 0)),
            pl.BlockSpec((_S, _HKV * _DH), lambda: (0, 0)),
        ],
        out_specs=pl.BlockSpec((32, 128), lambda: (0, 0)),
        out_shape=jax.ShapeDtypeStruct((32, 128), jnp.int32),
    )(q_last, k_emb)

    kv_i32 = lax.bitcast_convert_type(
        kv_emb.reshape(_S, _HKV * _DH, 2), jnp.int32)
    kv_sp_i32 = _sc_gather(dest32.reshape(_S), kv_i32)
    kv_sp = lax.bitcast_convert_type(
        kv_sp_i32[:_KPAD], jnp.bfloat16).reshape(_KPAD, 2 * _HKV * _DH)

    col = jnp.arange(_KPAD)
    bias_row = jnp.broadcast_to(
        jnp.where(col < _KSEL, 0.0, _NEG * _LOG2E).astype(jnp.float32)
        .reshape(1, _KPAD), (8, _KPAD))

    qblk = 1024
    out = pl.pallas_call(
        _attn_out_body,
        grid=(_S // qblk,),
        in_specs=[
            pl.BlockSpec((qblk, _H * _DH), lambda i: (i, 0)),
            pl.BlockSpec((_KPAD, 2 * _HKV * _DH), lambda i: (0, 0)),
            pl.BlockSpec((8, _KPAD), lambda i: (0, 0)),
            pl.BlockSpec((_DM, _DM), lambda i: (0, 0)),
        ],
        out_specs=pl.BlockSpec((qblk, _DM), lambda i: (i, 0)),
        out_shape=jax.ShapeDtypeStruct((_S, _DM), jnp.float32),
    )(q_emb, kv_sp, bias_row, wo)

    return out.reshape(_B, _S, _DM)
